# baseline (speedup 1.0000x reference)
import functools

import jax
import jax.numpy as jnp
from jax import lax
from jax.experimental import pallas as pl
from jax.experimental.pallas import tpu as pltpu
from jax.experimental.pallas import tpu_sc as plsc

_CHUNK = 128


@functools.lru_cache(maxsize=None)
def _build(B, V, D, nc, ns):
    nw = nc * ns
    b_per_w = B // nw
    nchunk = b_per_w // _CHUNK
    mesh = plsc.VectorSubcoreMesh(core_axis_name="c", subcore_axis_name="s")

    @functools.partial(
        pl.kernel,
        mesh=mesh,
        out_type=jax.ShapeDtypeStruct((B, D), jnp.float32),
        scratch_types=[
            pltpu.VMEM((nchunk, _CHUNK), jnp.int32),
            pltpu.VMEM((nchunk, _CHUNK, D), jnp.float32),
            pltpu.SemaphoreType.DMA,
        ],
        compiler_params=pltpu.CompilerParams(use_tc_tiling_on_sc=False),
    )
    def gather_kernel(idx_hbm, table_hbm, out_hbm, idx_v, rows_v, sem):
        wid = lax.axis_index("s") * nc + lax.axis_index("c")
        base = wid * b_per_w
        pltpu.sync_copy(idx_hbm.at[wid], idx_v)
        copies = []
        for j in range(nchunk):
            copies.append(
                pltpu.async_copy(table_hbm.at[idx_v.at[j]], rows_v.at[j], sem)
            )
        for c in copies:
            c.wait()
        for j in range(nchunk):
            pltpu.sync_copy(
                rows_v.at[j], out_hbm.at[pl.ds(base + j * _CHUNK, _CHUNK)]
            )

    return gather_kernel


def kernel(idx, context_hat):
    B = idx.shape[0]
    V, D = context_hat.shape
    info = plsc.get_sparse_core_info()
    nc, ns = info.num_cores, info.num_subcores
    nw = nc * ns
    idx_flat = idx.reshape(B).astype(jnp.int32)
    idx_3d = idx_flat.reshape(nw, B // (nw * _CHUNK), _CHUNK)
    return _build(B, V, D, nc, ns)(idx_3d, context_hat)
